# baseline (speedup 1.0000x reference)
import functools

import jax
import jax.numpy as jnp
from jax import lax
from jax.experimental import pallas as pl
from jax.experimental.pallas import tpu as pltpu
from jax.experimental.pallas import tpu_sc as plsc

_NUM_ENH = 8
_NC, _NS, _L = 2, 16, 16
_NW = _NC * _NS
_CHUNK = 256


def _sc_body(
    feat_hbm, out_hbm, buf0, buf1, tail, isem0, isem1, osem0, osem1, S, D
):
    wid = lax.axis_index("s") * _NC + lax.axis_index("c")
    n_lane = D // _L
    n_chunks = S // _CHUNK
    b_per_w = 4
    bufs = (buf0, buf1)
    isems, osems = (isem0, isem1), (osem0, osem1)
    pend_in = [None, None]
    pend_out = [None, None]

    sched = [
        (wid * b_per_w + bl, c)
        for bl in range(b_per_w)
        for c in range(n_chunks)
    ]

    def start_in(i):
        b, c = sched[i]
        k = i % 2
        dma = pltpu.make_async_copy(
            feat_hbm.at[pl.ds(b * S + c * _CHUNK, _CHUNK), :], bufs[k], isems[k]
        )
        dma.start()
        pend_in[k] = dma

    start_in(0)
    accs = tuple(jnp.zeros((_L,), jnp.float32) for _ in range(n_lane))
    for i, (b, c) in enumerate(sched):
        k = i % 2
        if i + 1 < len(sched):
            k2 = (i + 1) % 2
            if pend_out[k2] is not None:
                pend_out[k2].wait()
                pend_out[k2] = None
            start_in(i + 1)
        pend_in[k].wait()
        buf = bufs[k]
        out_dma = pltpu.make_async_copy(
            buf,
            out_hbm.at[pl.ds(b * (S + _NUM_ENH) + c * _CHUNK, _CHUNK), :],
            osems[k],
        )
        out_dma.start()
        pend_out[k] = out_dma

        def row_body(t, acc, _buf=buf, _n=n_lane):
            r = t * 4
            for u in range(4):
                acc = tuple(
                    acc[j] + _buf[r + u, pl.ds(_L * j, _L)] for j in range(_n)
                )
            return acc

        accs = lax.fori_loop(0, _CHUNK // 4, row_body, accs)
        if c == n_chunks - 1:
            for j in range(n_lane):
                for r in range(_NUM_ENH):
                    tail[r, pl.ds(_L * j, _L)] = accs[j]
            pltpu.sync_copy(
                tail, out_hbm.at[pl.ds(b * (S + _NUM_ENH) + S, _NUM_ENH), :]
            )
            accs = tuple(jnp.zeros((_L,), jnp.float32) for _ in range(n_lane))
    for k in (0, 1):
        if pend_out[k] is not None:
            pend_out[k].wait()


def kernel(features, alpha):
    del alpha
    B, S, D = features.shape
    feat2d = features.reshape(B * S, D)
    mesh = plsc.VectorSubcoreMesh(core_axis_name="c", subcore_axis_name="s")
    sc = pl.kernel(
        functools.partial(_sc_body, S=S, D=D),
        jax.ShapeDtypeStruct((B * (S + _NUM_ENH), D), features.dtype),
        mesh=mesh,
        scratch_types=[
            pltpu.VMEM((_CHUNK, D), jnp.float32),
            pltpu.VMEM((_CHUNK, D), jnp.float32),
            pltpu.VMEM((_NUM_ENH, D), jnp.float32),
            pltpu.SemaphoreType.DMA,
            pltpu.SemaphoreType.DMA,
            pltpu.SemaphoreType.DMA,
            pltpu.SemaphoreType.DMA,
        ],
    )
    out2d = sc(feat2d)
    return out2d.reshape(B, S + _NUM_ENH, D)


# speedup vs baseline: 1.0343x; 1.0343x over previous
import functools

import jax
import jax.numpy as jnp
from jax import lax
from jax.experimental import pallas as pl
from jax.experimental.pallas import tpu as pltpu
from jax.experimental.pallas import tpu_sc as plsc

_NUM_ENH = 8
_NC, _NS, _L = 2, 16, 16
_NW = _NC * _NS
_CHUNK = 256
_NBUF = 3


def _sc_body(feat_hbm, out_hbm, *refs, S, D):
    bufs = refs[:_NBUF]
    tail = refs[_NBUF]
    isems = refs[_NBUF + 1 : 2 * _NBUF + 1]
    osems = refs[2 * _NBUF + 1 : 3 * _NBUF + 1]
    wid = lax.axis_index("s") * _NC + lax.axis_index("c")
    n_lane = D // _L
    n_chunks = S // _CHUNK
    b_per_w = 4
    pend_in = [None] * _NBUF
    pend_out = [None] * _NBUF

    sched = [
        (wid * b_per_w + bl, c)
        for bl in range(b_per_w)
        for c in range(n_chunks)
    ]

    def start_in(i):
        b, c = sched[i]
        k = i % _NBUF
        dma = pltpu.make_async_copy(
            feat_hbm.at[pl.ds(b * S + c * _CHUNK, _CHUNK), :], bufs[k], isems[k]
        )
        dma.start()
        pend_in[k] = dma

    start_in(0)
    start_in(1)
    accs = tuple(jnp.zeros((_L,), jnp.float32) for _ in range(n_lane))
    for i, (b, c) in enumerate(sched):
        k = i % _NBUF
        if i + 2 < len(sched):
            k2 = (i + 2) % _NBUF
            if pend_out[k2] is not None:
                pend_out[k2].wait()
                pend_out[k2] = None
            start_in(i + 2)
        pend_in[k].wait()
        buf = bufs[k]
        out_dma = pltpu.make_async_copy(
            buf,
            out_hbm.at[pl.ds(b * (S + _NUM_ENH) + c * _CHUNK, _CHUNK), :],
            osems[k],
        )
        out_dma.start()
        pend_out[k] = out_dma

        def row_body(r, acc, _buf=buf, _n=n_lane):
            return tuple(acc[j] + _buf[r, pl.ds(_L * j, _L)] for j in range(_n))

        accs = lax.fori_loop(0, _CHUNK, row_body, accs)
        if c == n_chunks - 1:
            for j in range(n_lane):
                for r in range(_NUM_ENH):
                    tail[r, pl.ds(_L * j, _L)] = accs[j]
            pltpu.sync_copy(
                tail, out_hbm.at[pl.ds(b * (S + _NUM_ENH) + S, _NUM_ENH), :]
            )
            accs = tuple(jnp.zeros((_L,), jnp.float32) for _ in range(n_lane))
    for k in range(_NBUF):
        if pend_out[k] is not None:
            pend_out[k].wait()


def kernel(features, alpha):
    del alpha
    B, S, D = features.shape
    feat2d = features.reshape(B * S, D)
    mesh = plsc.VectorSubcoreMesh(core_axis_name="c", subcore_axis_name="s")
    sc = pl.kernel(
        functools.partial(_sc_body, S=S, D=D),
        jax.ShapeDtypeStruct((B * (S + _NUM_ENH), D), features.dtype),
        mesh=mesh,
        scratch_types=[pltpu.VMEM((_CHUNK, D), jnp.float32)] * _NBUF
        + [pltpu.VMEM((_NUM_ENH, D), jnp.float32)]
        + [pltpu.SemaphoreType.DMA] * (2 * _NBUF),
    )
    out2d = sc(feat2d)
    return out2d.reshape(B, S + _NUM_ENH, D)
